# initial kernel scaffold (unmeasured)
import jax
import jax.numpy as jnp
from jax import lax
from jax.experimental import pallas as pl
from jax.experimental.pallas import tpu as pltpu

N_DEV = 4


def kernel(x, w_mat):
    m_total, _ = x.shape
    _, n = w_mat.shape
    m_per = m_total // N_DEV

    def body(x_ref, w_ref, out_ref, send_buf, recv_buf, send_sems, recv_sems):
        my = lax.axis_index("i")
        left = (my - 1) % N_DEV
        right = (my + 1) % N_DEV

        barrier_sem = pltpu.get_barrier_semaphore()
        for nbr in (left, right):
            pl.semaphore_signal(
                barrier_sem, inc=1,
                device_id=(nbr,), device_id_type=pl.DeviceIdType.MESH,
            )
        pl.semaphore_wait(barrier_sem, 2)

        def partial_block(c):
            x_blk = x_ref[pl.ds(c * m_per, m_per), :]
            return jnp.dot(x_blk, w_ref[:, :], preferred_element_type=jnp.float32)

        for h in range(N_DEV - 1):
            c = (my - 1 - h) % N_DEV
            part = partial_block(c)
            if h > 0:
                part = part + recv_buf[h - 1].astype(jnp.float32)
            send_buf[h] = part.astype(jnp.bfloat16)
            rdma = pltpu.make_async_remote_copy(
                src_ref=send_buf.at[h],
                dst_ref=recv_buf.at[h],
                send_sem=send_sems.at[h],
                recv_sem=recv_sems.at[h],
                device_id=(right,),
                device_id_type=pl.DeviceIdType.MESH,
            )
            rdma.start()
            rdma.wait()

        y = partial_block(my) + recv_buf[N_DEV - 2].astype(jnp.float32)
        k = 0.7978845608028654
        out_ref[:, :] = 0.5 * y * (1.0 + jnp.tanh(k * (y + 0.044715 * y * y * y)))

    return pl.pallas_call(
        body,
        out_shape=jax.ShapeDtypeStruct((m_per, n), jnp.float32),
        in_specs=[
            pl.BlockSpec(memory_space=pltpu.VMEM),
            pl.BlockSpec(memory_space=pltpu.VMEM),
        ],
        out_specs=pl.BlockSpec(memory_space=pltpu.VMEM),
        scratch_shapes=[
            pltpu.VMEM((N_DEV - 1, m_per, n), jnp.bfloat16),
            pltpu.VMEM((N_DEV - 1, m_per, n), jnp.bfloat16),
            pltpu.SemaphoreType.DMA((N_DEV - 1,)),
            pltpu.SemaphoreType.DMA((N_DEV - 1,)),
        ],
        compiler_params=pltpu.CompilerParams(collective_id=0),
    )(x, w_mat)


# baseline (device time: 182016 ns/iter reference)
import jax
import jax.numpy as jnp
from jax import lax
from jax.experimental import pallas as pl
from jax.experimental.pallas import tpu as pltpu

N_DEV = 4


def kernel(x, w_mat):
    m_total, k_per = x.shape
    _, n = w_mat.shape
    m_per = m_total // N_DEV

    def body(x_ref, w_ref, out_ref, w_bf, send_buf, recv_buf, send_sems, recv_sems):
        my = lax.axis_index("i")
        left = (my - 1) % N_DEV
        right = (my + 1) % N_DEV

        barrier_sem = pltpu.get_barrier_semaphore()
        for nbr in (left, right):
            pl.semaphore_signal(
                barrier_sem, inc=1,
                device_id=(nbr,), device_id_type=pl.DeviceIdType.MESH,
            )
        pl.semaphore_wait(barrier_sem, 2)

        w_bf[:, :] = w_ref[:, :].astype(jnp.bfloat16)

        def partial_block(c):
            x_blk = x_ref[pl.ds(c * m_per, m_per), :].astype(jnp.bfloat16)
            return jnp.dot(x_blk, w_bf[:, :], preferred_element_type=jnp.float32)

        for h in range(N_DEV - 1):
            c = (my - 1 - h) % N_DEV
            part = partial_block(c)
            if h > 0:
                part = part + recv_buf[h - 1].astype(jnp.float32)
            send_buf[h] = part.astype(jnp.bfloat16)
            rdma = pltpu.make_async_remote_copy(
                src_ref=send_buf.at[h],
                dst_ref=recv_buf.at[h],
                send_sem=send_sems.at[h],
                recv_sem=recv_sems.at[h],
                device_id=(right,),
                device_id_type=pl.DeviceIdType.MESH,
            )
            rdma.start()
            rdma.wait()

        y = partial_block(my) + recv_buf[N_DEV - 2].astype(jnp.float32)
        k = 0.7978845608028654
        out_ref[:, :] = 0.5 * y * (1.0 + jnp.tanh(k * (y + 0.044715 * y * y * y)))

    return pl.pallas_call(
        body,
        out_shape=jax.ShapeDtypeStruct((m_per, n), jnp.float32),
        in_specs=[
            pl.BlockSpec(memory_space=pltpu.VMEM),
            pl.BlockSpec(memory_space=pltpu.VMEM),
        ],
        out_specs=pl.BlockSpec(memory_space=pltpu.VMEM),
        scratch_shapes=[
            pltpu.VMEM((k_per, n), jnp.bfloat16),
            pltpu.VMEM((N_DEV - 1, m_per, n), jnp.bfloat16),
            pltpu.VMEM((N_DEV - 1, m_per, n), jnp.bfloat16),
            pltpu.SemaphoreType.DMA((N_DEV - 1,)),
            pltpu.SemaphoreType.DMA((N_DEV - 1,)),
        ],
        compiler_params=pltpu.CompilerParams(
            collective_id=0,
            vmem_limit_bytes=100 * 1024 * 1024,
        ),
    )(x, w_mat)


# device time: 108366 ns/iter; 1.6796x vs baseline; 1.6796x over previous
import jax
import jax.numpy as jnp
from jax import lax
from jax.experimental import pallas as pl
from jax.experimental.pallas import tpu as pltpu

N_DEV = 4


def _gelu(y):
    k = 0.7978845608028654
    return 0.5 * y * (1.0 + jnp.tanh(k * (y + 0.044715 * y * y * y)))


def kernel(x, w_mat):
    m_total, k_per = x.shape
    _, n = w_mat.shape
    m_per = m_total // N_DEV
    n_half = n // 2

    def body(x_ref, w_ref, out_ref, w_bf,
             send_r, recv_r, send_l, recv_l,
             ss_r, rs_r, ss_l, rs_l):
        my = lax.axis_index("i")
        left = (my - 1) % N_DEV
        right = (my + 1) % N_DEV

        barrier_sem = pltpu.get_barrier_semaphore()
        for nbr in (left, right):
            pl.semaphore_signal(
                barrier_sem, inc=1,
                device_id=(nbr,), device_id_type=pl.DeviceIdType.MESH,
            )
        pl.semaphore_wait(barrier_sem, 2)

        w_bf[:, :] = w_ref[:, :].astype(jnp.bfloat16)

        def part(c, col0):
            x_blk = x_ref[pl.ds(c * m_per, m_per), :].astype(jnp.bfloat16)
            return jnp.dot(
                x_blk, w_bf[:, col0:col0 + n_half],
                preferred_element_type=jnp.float32,
            )

        def mk(src, dst, ssem, rsem, tgt):
            return pltpu.make_async_remote_copy(
                src_ref=src, dst_ref=dst, send_sem=ssem, recv_sem=rsem,
                device_id=(tgt,), device_id_type=pl.DeviceIdType.MESH,
            )

        descs_r = [mk(send_r.at[h % 2], recv_r.at[h], ss_r.at[h % 2], rs_r.at[h], right)
                   for h in range(N_DEV - 1)]
        descs_l = [mk(send_l.at[h % 2], recv_l.at[h], ss_l.at[h % 2], rs_l.at[h], left)
                   for h in range(N_DEV - 1)]

        send_r[0] = part((my - 1) % N_DEV, 0).astype(jnp.bfloat16)
        descs_r[0].start()
        send_l[0] = part((my + 1) % N_DEV, n_half).astype(jnp.bfloat16)
        descs_l[0].start()

        for h in range(N_DEV - 2):
            pr = part((my - 2 - h) % N_DEV, 0)
            pll = part((my + 2 + h) % N_DEV, n_half)
            if (h + 1) % 2 == 0:
                descs_r[h - 1].wait_send()
                descs_l[h - 1].wait_send()
            descs_r[h].wait_recv()
            send_r[(h + 1) % 2] = (pr + recv_r[h].astype(jnp.float32)).astype(jnp.bfloat16)
            descs_r[h + 1].start()
            descs_l[h].wait_recv()
            send_l[(h + 1) % 2] = (pll + recv_l[h].astype(jnp.float32)).astype(jnp.bfloat16)
            descs_l[h + 1].start()

        pr = part(my, 0)
        pll = part(my, n_half)
        descs_r[N_DEV - 2].wait_recv()
        out_ref[:, 0:n_half] = _gelu(pr + recv_r[N_DEV - 2].astype(jnp.float32))
        descs_l[N_DEV - 2].wait_recv()
        out_ref[:, n_half:n] = _gelu(pll + recv_l[N_DEV - 2].astype(jnp.float32))

        for h in range(N_DEV - 3, N_DEV - 1):
            descs_r[h].wait_send()
            descs_l[h].wait_send()

    return pl.pallas_call(
        body,
        out_shape=jax.ShapeDtypeStruct((m_per, n), jnp.float32),
        in_specs=[
            pl.BlockSpec(memory_space=pltpu.VMEM),
            pl.BlockSpec(memory_space=pltpu.VMEM),
        ],
        out_specs=pl.BlockSpec(memory_space=pltpu.VMEM),
        scratch_shapes=[
            pltpu.VMEM((k_per, n), jnp.bfloat16),
            pltpu.VMEM((2, m_per, n_half), jnp.bfloat16),
            pltpu.VMEM((N_DEV - 1, m_per, n_half), jnp.bfloat16),
            pltpu.VMEM((2, m_per, n_half), jnp.bfloat16),
            pltpu.VMEM((N_DEV - 1, m_per, n_half), jnp.bfloat16),
            pltpu.SemaphoreType.DMA((2,)),
            pltpu.SemaphoreType.DMA((N_DEV - 1,)),
            pltpu.SemaphoreType.DMA((2,)),
            pltpu.SemaphoreType.DMA((N_DEV - 1,)),
        ],
        compiler_params=pltpu.CompilerParams(
            collective_id=0,
            vmem_limit_bytes=110 * 1024 * 1024,
        ),
    )(x, w_mat)


# device time: 104580 ns/iter; 1.7404x vs baseline; 1.0362x over previous
import jax
import jax.numpy as jnp
from jax import lax
from jax.experimental import pallas as pl
from jax.experimental.pallas import tpu as pltpu

N_DEV = 4


def _gelu(y):
    k = 0.7978845608028654
    return 0.5 * y * (1.0 + jnp.tanh(k * (y + 0.044715 * y * y * y)))


def kernel(x, w_mat):
    m_total, k_per = x.shape
    _, n = w_mat.shape
    m_per = m_total // N_DEV
    n_half = n // 2
    m_half = m_per // 2

    def body(x_hbm, w_ref, out_ref, w_bf, xs, xb,
             send_r, recv_r, send_l, recv_l,
             lsem, ss_r, rs_r, ss_l, rs_l):
        my = lax.axis_index("i")
        left = (my - 1) % N_DEV
        right = (my + 1) % N_DEV

        barrier_sem = pltpu.get_barrier_semaphore()
        for nbr in (left, right):
            pl.semaphore_signal(
                barrier_sem, inc=1,
                device_id=(nbr,), device_id_type=pl.DeviceIdType.MESH,
            )
        pl.semaphore_wait(barrier_sem, 2)

        def stream(c, slot):
            return pltpu.make_async_copy(
                x_hbm.at[pl.ds(c * m_per, m_per), :], xs.at[slot], lsem.at[slot]
            )

        c_a = stream((my - 1) % N_DEV, 0)
        c_a.start()
        c_b = stream((my + 1) % N_DEV, 0 + 1)
        c_b.start()

        w_bf[:, :] = w_ref[:, :].astype(jnp.bfloat16)
        w0 = w_bf.at[:, 0:n_half]
        w1 = w_bf.at[:, n_half:n]

        def mk(src, dst, ssem, rsem, tgt):
            return pltpu.make_async_remote_copy(
                src_ref=src, dst_ref=dst, send_sem=ssem, recv_sem=rsem,
                device_id=(tgt,), device_id_type=pl.DeviceIdType.MESH,
            )

        d_r0a = mk(send_r.at[0, pl.ds(0, m_half)], recv_r.at[0, pl.ds(0, m_half)],
                   ss_r.at[0], rs_r.at[0], right)
        d_r0b = mk(send_r.at[0, pl.ds(m_half, m_half)], recv_r.at[0, pl.ds(m_half, m_half)],
                   ss_r.at[1], rs_r.at[1], right)
        d_r1 = mk(send_r.at[1], recv_r.at[1], ss_r.at[2], rs_r.at[2], right)
        d_r2 = mk(send_r.at[0], recv_r.at[2], ss_r.at[3], rs_r.at[3], right)
        d_l0a = mk(send_l.at[0, pl.ds(0, m_half)], recv_l.at[0, pl.ds(0, m_half)],
                   ss_l.at[0], rs_l.at[0], left)
        d_l0b = mk(send_l.at[0, pl.ds(m_half, m_half)], recv_l.at[0, pl.ds(m_half, m_half)],
                   ss_l.at[1], rs_l.at[1], left)
        d_l1 = mk(send_l.at[1], recv_l.at[1], ss_l.at[2], rs_l.at[2], left)
        d_l2 = mk(send_l.at[0], recv_l.at[2], ss_l.at[3], rs_l.at[3], left)

        def dot_bf(a, wref):
            return jnp.dot(
                a, wref[:, :], preferred_element_type=jnp.float32
            ).astype(jnp.bfloat16)

        c_a.wait()
        xb[0] = xs[0].astype(jnp.bfloat16)
        send_r[0, 0:m_half] = dot_bf(xb[0, 0:m_half], w0)
        d_r0a.start()
        send_r[0, m_half:m_per] = dot_bf(xb[0, m_half:m_per], w0)
        d_r0b.start()
        c_b.wait()
        xb[1] = xs[1].astype(jnp.bfloat16)
        send_l[0, 0:m_half] = dot_bf(xb[1, 0:m_half], w1)
        d_l0a.start()
        send_l[0, m_half:m_per] = dot_bf(xb[1, m_half:m_per], w1)
        d_l0b.start()

        c_c = stream((my + 2) % N_DEV, 0)
        c_c.start()
        c_d = stream(my, 1)
        c_d.start()

        c_c.wait()
        xc = xs[0].astype(jnp.bfloat16)
        pr = dot_bf(xc, w0)
        pll = dot_bf(xc, w1)
        d_r0a.wait_recv()
        d_r0b.wait_recv()
        send_r[1] = pr + recv_r[0]
        d_r1.start()
        d_l0a.wait_recv()
        d_l0b.wait_recv()
        send_l[1] = pll + recv_l[0]
        d_l1.start()

        pr = dot_bf(xb[1], w0)
        pll = dot_bf(xb[0], w1)
        d_r0a.wait_send()
        d_r0b.wait_send()
        d_l0a.wait_send()
        d_l0b.wait_send()
        d_r1.wait_recv()
        send_r[0] = pr + recv_r[1]
        d_r2.start()
        d_l1.wait_recv()
        send_l[0] = pll + recv_l[1]
        d_l2.start()

        c_d.wait()
        xm = xs[1].astype(jnp.bfloat16)
        prf = jnp.dot(xm, w0[:, :], preferred_element_type=jnp.float32)
        pllf = jnp.dot(xm, w1[:, :], preferred_element_type=jnp.float32)
        d_r2.wait_recv()
        out_ref[:, 0:n_half] = _gelu(prf + recv_r[2].astype(jnp.float32))
        d_l2.wait_recv()
        out_ref[:, n_half:n] = _gelu(pllf + recv_l[2].astype(jnp.float32))

        d_r1.wait_send()
        d_l1.wait_send()
        d_r2.wait_send()
        d_l2.wait_send()

    return pl.pallas_call(
        body,
        out_shape=jax.ShapeDtypeStruct((m_per, n), jnp.float32),
        in_specs=[
            pl.BlockSpec(memory_space=pltpu.MemorySpace.HBM),
            pl.BlockSpec(memory_space=pltpu.VMEM),
        ],
        out_specs=pl.BlockSpec(memory_space=pltpu.VMEM),
        scratch_shapes=[
            pltpu.VMEM((k_per, n), jnp.bfloat16),
            pltpu.VMEM((2, m_per, k_per), jnp.float32),
            pltpu.VMEM((2, m_per, k_per), jnp.bfloat16),
            pltpu.VMEM((2, m_per, n_half), jnp.bfloat16),
            pltpu.VMEM((N_DEV - 1, m_per, n_half), jnp.bfloat16),
            pltpu.VMEM((2, m_per, n_half), jnp.bfloat16),
            pltpu.VMEM((N_DEV - 1, m_per, n_half), jnp.bfloat16),
            pltpu.SemaphoreType.DMA((2,)),
            pltpu.SemaphoreType.DMA((4,)),
            pltpu.SemaphoreType.DMA((4,)),
            pltpu.SemaphoreType.DMA((4,)),
            pltpu.SemaphoreType.DMA((4,)),
        ],
        compiler_params=pltpu.CompilerParams(
            collective_id=0,
            vmem_limit_bytes=110 * 1024 * 1024,
        ),
    )(x, w_mat)


# device time: 89955 ns/iter; 2.0234x vs baseline; 1.1626x over previous
import jax
import jax.numpy as jnp
from jax import lax
from jax.experimental import pallas as pl
from jax.experimental.pallas import tpu as pltpu

N_DEV = 4


def _gelu(y):
    k = 0.7978845608028654
    return 0.5 * y * (1.0 + jnp.tanh(k * (y + 0.044715 * y * y * y)))


def kernel(x, w_mat):
    m_total, k_per = x.shape
    _, n = w_mat.shape
    m_per = m_total // N_DEV
    n_half = n // 2
    mh = m_per // 2

    def body(x_hbm, w_ref, out_ref, w_bf, xs, xb,
             send_r, recv_r, send_l, recv_l,
             lsem, ss_r, rs_r, ss_l, rs_l):
        my = lax.axis_index("i")
        left = (my - 1) % N_DEV
        right = (my + 1) % N_DEV

        barrier_sem = pltpu.get_barrier_semaphore()
        for nbr in (left, right):
            pl.semaphore_signal(
                barrier_sem, inc=1,
                device_id=(nbr,), device_id_type=pl.DeviceIdType.MESH,
            )
        pl.semaphore_wait(barrier_sem, 2)

        def stream(c, slot):
            return pltpu.make_async_copy(
                x_hbm.at[pl.ds(c * m_per, m_per), :], xs.at[slot], lsem.at[slot]
            )

        c_a = stream((my - 1) % N_DEV, 0)
        c_a.start()
        c_b = stream((my + 1) % N_DEV, 1)
        c_b.start()

        w_bf[:, :] = w_ref[:, :].astype(jnp.bfloat16)
        w0 = w_bf.at[:, 0:n_half]
        w1 = w_bf.at[:, n_half:n]

        def mk(src, dst, ssem, rsem, tgt):
            return pltpu.make_async_remote_copy(
                src_ref=src, dst_ref=dst, send_sem=ssem, recv_sem=rsem,
                device_id=(tgt,), device_id_type=pl.DeviceIdType.MESH,
            )

        rows = (pl.ds(0, mh), pl.ds(mh, mh))
        d_r = [[mk(send_r.at[h % 2, rows[s]], recv_r.at[h, rows[s]],
                   ss_r.at[2 * h + s], rs_r.at[2 * h + s], right)
                for s in range(2)] for h in range(N_DEV - 1)]
        d_l = [[mk(send_l.at[h % 2, rows[s]], recv_l.at[h, rows[s]],
                   ss_l.at[2 * h + s], rs_l.at[2 * h + s], left)
                for s in range(2)] for h in range(N_DEV - 1)]

        def dot_bf(a, wref):
            return jnp.dot(
                a, wref[:, :], preferred_element_type=jnp.float32
            ).astype(jnp.bfloat16)

        c_a.wait()
        xb[0] = xs[0].astype(jnp.bfloat16)
        send_r[0, 0:mh] = dot_bf(xb[0, 0:mh], w0)
        d_r[0][0].start()
        c_b.wait()
        xb[1] = xs[1].astype(jnp.bfloat16)
        send_l[0, 0:mh] = dot_bf(xb[1, 0:mh], w1)
        d_l[0][0].start()
        send_r[0, mh:m_per] = dot_bf(xb[0, mh:m_per], w0)
        d_r[0][1].start()
        send_l[0, mh:m_per] = dot_bf(xb[1, mh:m_per], w1)
        d_l[0][1].start()

        c_c = stream((my + 2) % N_DEV, 0)
        c_c.start()
        c_d = stream(my, 1)

        c_c.wait()
        c_d.start()
        xc = xs[0].astype(jnp.bfloat16)
        pr = dot_bf(xc[0:mh], w0)
        pll = dot_bf(xc[0:mh], w1)
        d_r[0][0].wait_recv()
        send_r[1, 0:mh] = pr + recv_r[0, 0:mh]
        d_r[1][0].start()
        d_l[0][0].wait_recv()
        send_l[1, 0:mh] = pll + recv_l[0, 0:mh]
        d_l[1][0].start()
        pr = dot_bf(xc[mh:m_per], w0)
        pll = dot_bf(xc[mh:m_per], w1)
        d_r[0][1].wait_recv()
        send_r[1, mh:m_per] = pr + recv_r[0, mh:m_per]
        d_r[1][1].start()
        d_l[0][1].wait_recv()
        send_l[1, mh:m_per] = pll + recv_l[0, mh:m_per]
        d_l[1][1].start()

        pr = dot_bf(xb[1, 0:mh], w0)
        pll = dot_bf(xb[0, 0:mh], w1)
        d_r[0][0].wait_send()
        d_r[0][1].wait_send()
        d_l[0][0].wait_send()
        d_l[0][1].wait_send()
        d_r[1][0].wait_recv()
        send_r[0, 0:mh] = pr + recv_r[1, 0:mh]
        d_r[2][0].start()
        d_l[1][0].wait_recv()
        send_l[0, 0:mh] = pll + recv_l[1, 0:mh]
        d_l[2][0].start()
        pr = dot_bf(xb[1, mh:m_per], w0)
        pll = dot_bf(xb[0, mh:m_per], w1)
        d_r[1][1].wait_recv()
        send_r[0, mh:m_per] = pr + recv_r[1, mh:m_per]
        d_r[2][1].start()
        d_l[1][1].wait_recv()
        send_l[0, mh:m_per] = pll + recv_l[1, mh:m_per]
        d_l[2][1].start()

        c_d.wait()
        xm = xs[1].astype(jnp.bfloat16)
        prf = jnp.dot(xm[0:mh], w0[:, :], preferred_element_type=jnp.float32)
        plf = jnp.dot(xm[0:mh], w1[:, :], preferred_element_type=jnp.float32)
        d_r[2][0].wait_recv()
        out_ref[0:mh, 0:n_half] = _gelu(
            prf + recv_r[2, 0:mh].astype(jnp.float32)).astype(jnp.bfloat16)
        d_l[2][0].wait_recv()
        out_ref[0:mh, n_half:n] = _gelu(
            plf + recv_l[2, 0:mh].astype(jnp.float32)).astype(jnp.bfloat16)
        prf = jnp.dot(xm[mh:m_per], w0[:, :], preferred_element_type=jnp.float32)
        plf = jnp.dot(xm[mh:m_per], w1[:, :], preferred_element_type=jnp.float32)
        d_r[2][1].wait_recv()
        out_ref[mh:m_per, 0:n_half] = _gelu(
            prf + recv_r[2, mh:m_per].astype(jnp.float32)).astype(jnp.bfloat16)
        d_l[2][1].wait_recv()
        out_ref[mh:m_per, n_half:n] = _gelu(
            plf + recv_l[2, mh:m_per].astype(jnp.float32)).astype(jnp.bfloat16)

        for h in (1, 2):
            for s in range(2):
                d_r[h][s].wait_send()
                d_l[h][s].wait_send()

    return pl.pallas_call(
        body,
        out_shape=jax.ShapeDtypeStruct((m_per, n), jnp.bfloat16),
        in_specs=[
            pl.BlockSpec(memory_space=pltpu.MemorySpace.HBM),
            pl.BlockSpec(memory_space=pltpu.VMEM),
        ],
        out_specs=pl.BlockSpec(memory_space=pltpu.VMEM),
        scratch_shapes=[
            pltpu.VMEM((k_per, n), jnp.bfloat16),
            pltpu.VMEM((2, m_per, k_per), jnp.float32),
            pltpu.VMEM((2, m_per, k_per), jnp.bfloat16),
            pltpu.VMEM((2, m_per, n_half), jnp.bfloat16),
            pltpu.VMEM((N_DEV - 1, m_per, n_half), jnp.bfloat16),
            pltpu.VMEM((2, m_per, n_half), jnp.bfloat16),
            pltpu.VMEM((N_DEV - 1, m_per, n_half), jnp.bfloat16),
            pltpu.SemaphoreType.DMA((2,)),
            pltpu.SemaphoreType.DMA((6,)),
            pltpu.SemaphoreType.DMA((6,)),
            pltpu.SemaphoreType.DMA((6,)),
            pltpu.SemaphoreType.DMA((6,)),
        ],
        compiler_params=pltpu.CompilerParams(
            collective_id=0,
            vmem_limit_bytes=110 * 1024 * 1024,
        ),
    )(x, w_mat)


# device time: 87679 ns/iter; 2.0759x vs baseline; 1.0260x over previous
import jax
import jax.numpy as jnp
from jax import lax
from jax.experimental import pallas as pl
from jax.experimental.pallas import tpu as pltpu

N_DEV = 4


def _gelu(y):
    k = 0.7978845608028654
    return 0.5 * y * (1.0 + jnp.tanh(k * (y + 0.044715 * y * y * y)))


def kernel(x, w_mat):
    m_total, k_per = x.shape
    _, n = w_mat.shape
    m_per = m_total // N_DEV
    n_half = n // 2
    mh = m_per // 2

    def body(x_hbm, w_hbm, out_ref, ws, w_bf, xs, xb,
             send_r, recv_r, send_l, recv_l,
             lsem, ss_r, rs_r, ss_l, rs_l):
        my = lax.axis_index("i")
        left = (my - 1) % N_DEV
        right = (my + 1) % N_DEV

        barrier_sem = pltpu.get_barrier_semaphore()
        for nbr in (left, right):
            pl.semaphore_signal(
                barrier_sem, inc=1,
                device_id=(nbr,), device_id_type=pl.DeviceIdType.MESH,
            )

        def stream(c, slot):
            return pltpu.make_async_copy(
                x_hbm.at[pl.ds(c * m_per, m_per), :], xs.at[slot], lsem.at[slot]
            )

        c_w0 = pltpu.make_async_copy(
            w_hbm.at[:, pl.ds(0, n_half)], ws.at[0], lsem.at[2])
        c_w0.start()
        c_a = stream((my - 1) % N_DEV, 0)
        c_a.start()
        c_b = stream((my + 1) % N_DEV, 1)
        c_b.start()
        c_w1 = pltpu.make_async_copy(
            w_hbm.at[:, pl.ds(n_half, n_half)], ws.at[1], lsem.at[3])
        c_w1.start()

        w0 = w_bf.at[:, 0:n_half]
        w1 = w_bf.at[:, n_half:n]

        def mk(src, dst, ssem, rsem, tgt):
            return pltpu.make_async_remote_copy(
                src_ref=src, dst_ref=dst, send_sem=ssem, recv_sem=rsem,
                device_id=(tgt,), device_id_type=pl.DeviceIdType.MESH,
            )

        rows = (pl.ds(0, mh), pl.ds(mh, mh))
        d_r = [[mk(send_r.at[h % 2, rows[s]], recv_r.at[h, rows[s]],
                   ss_r.at[2 * h + s], rs_r.at[2 * h + s], right)
                for s in range(2)] for h in range(N_DEV - 1)]
        d_l = [[mk(send_l.at[h % 2, rows[s]], recv_l.at[h, rows[s]],
                   ss_l.at[2 * h + s], rs_l.at[2 * h + s], left)
                for s in range(2)] for h in range(N_DEV - 1)]

        def dot_bf(a, wref):
            return jnp.dot(
                a, wref[:, :], preferred_element_type=jnp.float32
            ).astype(jnp.bfloat16)

        c_w0.wait()
        w_bf[:, 0:n_half] = ws[0].astype(jnp.bfloat16)
        c_a.wait()
        xb[0] = xs[0].astype(jnp.bfloat16)
        send_r[0, 0:mh] = dot_bf(xb[0, 0:mh], w0)
        pl.semaphore_wait(barrier_sem, 2)
        d_r[0][0].start()
        c_w1.wait()
        w_bf[:, n_half:n] = ws[1].astype(jnp.bfloat16)
        c_b.wait()
        xb[1] = xs[1].astype(jnp.bfloat16)
        send_l[0, 0:mh] = dot_bf(xb[1, 0:mh], w1)
        d_l[0][0].start()
        send_r[0, mh:m_per] = dot_bf(xb[0, mh:m_per], w0)
        d_r[0][1].start()
        send_l[0, mh:m_per] = dot_bf(xb[1, mh:m_per], w1)
        d_l[0][1].start()

        c_c = stream((my + 2) % N_DEV, 0)
        c_c.start()
        c_d = stream(my, 1)

        c_c.wait()
        c_d.start()
        xc = xs[0].astype(jnp.bfloat16)
        pr = dot_bf(xc[0:mh], w0)
        pll = dot_bf(xc[0:mh], w1)
        d_r[0][0].wait_recv()
        send_r[1, 0:mh] = pr + recv_r[0, 0:mh]
        d_r[1][0].start()
        d_l[0][0].wait_recv()
        send_l[1, 0:mh] = pll + recv_l[0, 0:mh]
        d_l[1][0].start()
        pr = dot_bf(xc[mh:m_per], w0)
        pll = dot_bf(xc[mh:m_per], w1)
        d_r[0][1].wait_recv()
        send_r[1, mh:m_per] = pr + recv_r[0, mh:m_per]
        d_r[1][1].start()
        d_l[0][1].wait_recv()
        send_l[1, mh:m_per] = pll + recv_l[0, mh:m_per]
        d_l[1][1].start()

        pr = dot_bf(xb[1, 0:mh], w0)
        pll = dot_bf(xb[0, 0:mh], w1)
        d_r[0][0].wait_send()
        d_r[0][1].wait_send()
        d_l[0][0].wait_send()
        d_l[0][1].wait_send()
        d_r[1][0].wait_recv()
        send_r[0, 0:mh] = pr + recv_r[1, 0:mh]
        d_r[2][0].start()
        d_l[1][0].wait_recv()
        send_l[0, 0:mh] = pll + recv_l[1, 0:mh]
        d_l[2][0].start()
        pr = dot_bf(xb[1, mh:m_per], w0)
        pll = dot_bf(xb[0, mh:m_per], w1)
        d_r[1][1].wait_recv()
        send_r[0, mh:m_per] = pr + recv_r[1, mh:m_per]
        d_r[2][1].start()
        d_l[1][1].wait_recv()
        send_l[0, mh:m_per] = pll + recv_l[1, mh:m_per]
        d_l[2][1].start()

        c_d.wait()
        xm = xs[1].astype(jnp.bfloat16)
        prf = jnp.dot(xm[0:mh], w0[:, :], preferred_element_type=jnp.float32)
        plf = jnp.dot(xm[0:mh], w1[:, :], preferred_element_type=jnp.float32)
        d_r[2][0].wait_recv()
        out_ref[0:mh, 0:n_half] = _gelu(
            prf + recv_r[2, 0:mh].astype(jnp.float32)).astype(jnp.bfloat16)
        d_l[2][0].wait_recv()
        out_ref[0:mh, n_half:n] = _gelu(
            plf + recv_l[2, 0:mh].astype(jnp.float32)).astype(jnp.bfloat16)
        prf = jnp.dot(xm[mh:m_per], w0[:, :], preferred_element_type=jnp.float32)
        plf = jnp.dot(xm[mh:m_per], w1[:, :], preferred_element_type=jnp.float32)
        d_r[2][1].wait_recv()
        out_ref[mh:m_per, 0:n_half] = _gelu(
            prf + recv_r[2, mh:m_per].astype(jnp.float32)).astype(jnp.bfloat16)
        d_l[2][1].wait_recv()
        out_ref[mh:m_per, n_half:n] = _gelu(
            plf + recv_l[2, mh:m_per].astype(jnp.float32)).astype(jnp.bfloat16)

        for h in (1, 2):
            for s in range(2):
                d_r[h][s].wait_send()
                d_l[h][s].wait_send()

    return pl.pallas_call(
        body,
        out_shape=jax.ShapeDtypeStruct((m_per, n), jnp.bfloat16),
        in_specs=[
            pl.BlockSpec(memory_space=pltpu.MemorySpace.HBM),
            pl.BlockSpec(memory_space=pltpu.MemorySpace.HBM),
        ],
        out_specs=pl.BlockSpec(memory_space=pltpu.VMEM),
        scratch_shapes=[
            pltpu.VMEM((2, k_per, n_half), jnp.float32),
            pltpu.VMEM((k_per, n), jnp.bfloat16),
            pltpu.VMEM((2, m_per, k_per), jnp.float32),
            pltpu.VMEM((2, m_per, k_per), jnp.bfloat16),
            pltpu.VMEM((2, m_per, n_half), jnp.bfloat16),
            pltpu.VMEM((N_DEV - 1, m_per, n_half), jnp.bfloat16),
            pltpu.VMEM((2, m_per, n_half), jnp.bfloat16),
            pltpu.VMEM((N_DEV - 1, m_per, n_half), jnp.bfloat16),
            pltpu.SemaphoreType.DMA((4,)),
            pltpu.SemaphoreType.DMA((6,)),
            pltpu.SemaphoreType.DMA((6,)),
            pltpu.SemaphoreType.DMA((6,)),
            pltpu.SemaphoreType.DMA((6,)),
        ],
        compiler_params=pltpu.CompilerParams(
            collective_id=0,
            vmem_limit_bytes=110 * 1024 * 1024,
        ),
    )(x, w_mat)


# device time: 86277 ns/iter; 2.1097x vs baseline; 1.0162x over previous
import jax
import jax.numpy as jnp
from jax import lax
from jax.experimental import pallas as pl
from jax.experimental.pallas import tpu as pltpu

N_DEV = 4


def _gelu(y):
    k = 0.7978845608028654
    return 0.5 * y * (1.0 + jnp.tanh(k * (y + 0.044715 * y * y * y)))


def kernel(x, w_mat):
    m_total, k_per = x.shape
    _, n = w_mat.shape
    m_per = m_total // N_DEV
    n_half = n // 2
    mh = m_per // 2

    def body(x_hbm, w_hbm, out_ref, ws, w_bf, xs, xb,
             send_r, recv_r, send_l, recv_l,
             lsem, ss_r, rs_r, ss_l, rs_l):
        my = lax.axis_index("i")
        left = (my - 1) % N_DEV
        right = (my + 1) % N_DEV

        barrier_sem = pltpu.get_barrier_semaphore()
        for nbr in (left, right):
            pl.semaphore_signal(
                barrier_sem, inc=1,
                device_id=(nbr,), device_id_type=pl.DeviceIdType.MESH,
            )

        def stream(c, slot, sem):
            return pltpu.make_async_copy(
                x_hbm.at[pl.ds(c * m_per, m_per), :], xs.at[slot], lsem.at[sem]
            )

        def stream_half(c, s, slot, sem):
            return pltpu.make_async_copy(
                x_hbm.at[pl.ds(c * m_per + s * mh, mh), :],
                xs.at[slot, pl.ds(s * mh, mh)], lsem.at[sem],
            )

        c_w0 = pltpu.make_async_copy(
            w_hbm.at[:, pl.ds(0, n_half)], ws.at[0], lsem.at[2])
        c_w0.start()
        c_a0 = stream_half((my - 1) % N_DEV, 0, 0, 0)
        c_a0.start()
        c_b0 = stream_half((my + 1) % N_DEV, 0, 1, 1)
        c_b0.start()
        c_w1 = pltpu.make_async_copy(
            w_hbm.at[:, pl.ds(n_half, n_half)], ws.at[1], lsem.at[3])
        c_w1.start()
        c_a1 = stream_half((my - 1) % N_DEV, 1, 0, 4)
        c_a1.start()
        c_b1 = stream_half((my + 1) % N_DEV, 1, 1, 5)
        c_b1.start()

        w0 = w_bf.at[:, 0:n_half]
        w1 = w_bf.at[:, n_half:n]

        def mk(src, dst, ssem, rsem, tgt):
            return pltpu.make_async_remote_copy(
                src_ref=src, dst_ref=dst, send_sem=ssem, recv_sem=rsem,
                device_id=(tgt,), device_id_type=pl.DeviceIdType.MESH,
            )

        rows = (pl.ds(0, mh), pl.ds(mh, mh))
        d_r = [[mk(send_r.at[h % 2, rows[s]], recv_r.at[h, rows[s]],
                   ss_r.at[2 * h + s], rs_r.at[2 * h + s], right)
                for s in range(2)] for h in range(N_DEV - 1)]
        d_l = [[mk(send_l.at[h % 2, rows[s]], recv_l.at[h, rows[s]],
                   ss_l.at[2 * h + s], rs_l.at[2 * h + s], left)
                for s in range(2)] for h in range(N_DEV - 1)]

        def dot_bf(a, wref):
            return jnp.dot(
                a, wref[:, :], preferred_element_type=jnp.float32
            ).astype(jnp.bfloat16)

        c_w0.wait()
        w_bf[:, 0:n_half] = ws[0].astype(jnp.bfloat16)
        c_a0.wait()
        xb[0, 0:mh] = xs[0, 0:mh].astype(jnp.bfloat16)
        send_r[0, 0:mh] = dot_bf(xb[0, 0:mh], w0)
        pl.semaphore_wait(barrier_sem, 2)
        d_r[0][0].start()
        c_w1.wait()
        w_bf[:, n_half:n] = ws[1].astype(jnp.bfloat16)
        c_b0.wait()
        xb[1, 0:mh] = xs[1, 0:mh].astype(jnp.bfloat16)
        send_l[0, 0:mh] = dot_bf(xb[1, 0:mh], w1)
        d_l[0][0].start()
        c_a1.wait()
        xb[0, mh:m_per] = xs[0, mh:m_per].astype(jnp.bfloat16)
        send_r[0, mh:m_per] = dot_bf(xb[0, mh:m_per], w0)
        d_r[0][1].start()
        c_b1.wait()
        xb[1, mh:m_per] = xs[1, mh:m_per].astype(jnp.bfloat16)
        send_l[0, mh:m_per] = dot_bf(xb[1, mh:m_per], w1)
        d_l[0][1].start()

        c_c = stream((my + 2) % N_DEV, 0, 0)
        c_c.start()
        c_d = stream(my, 1, 1)

        c_c.wait()
        c_d.start()
        xc = xs[0].astype(jnp.bfloat16)
        pr = dot_bf(xc[0:mh], w0)
        pll = dot_bf(xc[0:mh], w1)
        d_r[0][0].wait_recv()
        send_r[1, 0:mh] = pr + recv_r[0, 0:mh]
        d_r[1][0].start()
        d_l[0][0].wait_recv()
        send_l[1, 0:mh] = pll + recv_l[0, 0:mh]
        d_l[1][0].start()
        pr = dot_bf(xc[mh:m_per], w0)
        pll = dot_bf(xc[mh:m_per], w1)
        d_r[0][1].wait_recv()
        send_r[1, mh:m_per] = pr + recv_r[0, mh:m_per]
        d_r[1][1].start()
        d_l[0][1].wait_recv()
        send_l[1, mh:m_per] = pll + recv_l[0, mh:m_per]
        d_l[1][1].start()

        pr = dot_bf(xb[1, 0:mh], w0)
        pll = dot_bf(xb[0, 0:mh], w1)
        d_r[0][0].wait_send()
        d_r[0][1].wait_send()
        d_l[0][0].wait_send()
        d_l[0][1].wait_send()
        d_r[1][0].wait_recv()
        send_r[0, 0:mh] = pr + recv_r[1, 0:mh]
        d_r[2][0].start()
        d_l[1][0].wait_recv()
        send_l[0, 0:mh] = pll + recv_l[1, 0:mh]
        d_l[2][0].start()
        pr = dot_bf(xb[1, mh:m_per], w0)
        pll = dot_bf(xb[0, mh:m_per], w1)
        d_r[1][1].wait_recv()
        send_r[0, mh:m_per] = pr + recv_r[1, mh:m_per]
        d_r[2][1].start()
        d_l[1][1].wait_recv()
        send_l[0, mh:m_per] = pll + recv_l[1, mh:m_per]
        d_l[2][1].start()

        c_d.wait()
        xm = xs[1].astype(jnp.bfloat16)
        prf = jnp.dot(xm[0:mh], w0[:, :], preferred_element_type=jnp.float32)
        plf = jnp.dot(xm[0:mh], w1[:, :], preferred_element_type=jnp.float32)
        d_r[2][0].wait_recv()
        out_ref[0:mh, 0:n_half] = _gelu(
            prf + recv_r[2, 0:mh].astype(jnp.float32)).astype(jnp.bfloat16)
        d_l[2][0].wait_recv()
        out_ref[0:mh, n_half:n] = _gelu(
            plf + recv_l[2, 0:mh].astype(jnp.float32)).astype(jnp.bfloat16)
        prf = jnp.dot(xm[mh:m_per], w0[:, :], preferred_element_type=jnp.float32)
        plf = jnp.dot(xm[mh:m_per], w1[:, :], preferred_element_type=jnp.float32)
        d_r[2][1].wait_recv()
        out_ref[mh:m_per, 0:n_half] = _gelu(
            prf + recv_r[2, mh:m_per].astype(jnp.float32)).astype(jnp.bfloat16)
        d_l[2][1].wait_recv()
        out_ref[mh:m_per, n_half:n] = _gelu(
            plf + recv_l[2, mh:m_per].astype(jnp.float32)).astype(jnp.bfloat16)

        for h in (1, 2):
            for s in range(2):
                d_r[h][s].wait_send()
                d_l[h][s].wait_send()

    return pl.pallas_call(
        body,
        out_shape=jax.ShapeDtypeStruct((m_per, n), jnp.bfloat16),
        in_specs=[
            pl.BlockSpec(memory_space=pltpu.MemorySpace.HBM),
            pl.BlockSpec(memory_space=pltpu.MemorySpace.HBM),
        ],
        out_specs=pl.BlockSpec(memory_space=pltpu.VMEM),
        scratch_shapes=[
            pltpu.VMEM((2, k_per, n_half), jnp.float32),
            pltpu.VMEM((k_per, n), jnp.bfloat16),
            pltpu.VMEM((2, m_per, k_per), jnp.float32),
            pltpu.VMEM((2, m_per, k_per), jnp.bfloat16),
            pltpu.VMEM((2, m_per, n_half), jnp.bfloat16),
            pltpu.VMEM((N_DEV - 1, m_per, n_half), jnp.bfloat16),
            pltpu.VMEM((2, m_per, n_half), jnp.bfloat16),
            pltpu.VMEM((N_DEV - 1, m_per, n_half), jnp.bfloat16),
            pltpu.SemaphoreType.DMA((6,)),
            pltpu.SemaphoreType.DMA((6,)),
            pltpu.SemaphoreType.DMA((6,)),
            pltpu.SemaphoreType.DMA((6,)),
            pltpu.SemaphoreType.DMA((6,)),
        ],
        compiler_params=pltpu.CompilerParams(
            collective_id=0,
            vmem_limit_bytes=110 * 1024 * 1024,
        ),
    )(x, w_mat)
